# baseline (speedup 1.0000x reference)
import jax
import jax.numpy as jnp
from jax import lax
from jax.experimental import pallas as pl
from jax.experimental.pallas import tpu as pltpu
from jax.experimental.pallas import tpu_sc as plsc

N = 10000
E = 320000
NC = 2
NS = 16
NW = NC * NS
K = 128
EPT = E // NW
NCHUNK = 80
EPTP = NCHUNK * K
NPAD = 10112
RPT = NPAD // NS
D = 128
BM = 400
F32 = jnp.float32

_MESH = plsc.VectorSubcoreMesh(core_axis_name="c", subcore_axis_name="s")


def _deg_body(src_hbm, dst_hbm, out_hbm, sidx_all, didx_all, ones_v, zrow_v,
              acc_out, acc_in, sem0, sem1):
    c = lax.axis_index("c")
    s = lax.axis_index("s")
    wid = s * NC + c

    @pl.loop(0, K, step=16)
    def _(i):
        ones_v[pl.ds(i, 16)] = jnp.ones((16,), F32)

    @pl.loop(0, RPT + 8, step=16)
    def _(i):
        zrow_v[pl.ds(i, 16)] = jnp.zeros((16,), F32)

    pltpu.sync_copy(src_hbm.at[wid], sidx_all)
    pltpu.sync_copy(dst_hbm.at[wid], didx_all)

    r0 = s * RPT
    pltpu.sync_copy(zrow_v.at[pl.ds(0, RPT)], acc_out.at[pl.ds(r0, RPT)])
    pltpu.sync_copy(zrow_v.at[pl.ds(0, RPT)], acc_in.at[pl.ds(r0, RPT)])
    plsc.subcore_barrier()

    def fire(j):
        pltpu.async_copy(ones_v, acc_out.at[sidx_all.at[j]], sem0, add=True)
        pltpu.async_copy(ones_v, acc_in.at[didx_all.at[j]], sem1, add=True)

    def drain(j):
        pltpu.make_async_copy(ones_v, acc_out.at[sidx_all.at[j]], sem0).wait()
        pltpu.make_async_copy(ones_v, acc_in.at[didx_all.at[j]], sem1).wait()

    fire(0)

    @pl.loop(1, NCHUNK)
    def _(j):
        fire(j)
        drain(j - 1)

    drain(NCHUNK - 1)

    plsc.subcore_barrier()
    obase = c * 2 * NPAD

    pltpu.sync_copy(acc_out.at[pl.ds(r0, RPT)], zrow_v.at[pl.ds(0, RPT)])
    pltpu.sync_copy(zrow_v.at[pl.ds(0, RPT)], out_hbm.at[pl.ds(obase + r0, RPT)])
    pltpu.sync_copy(acc_in.at[pl.ds(r0, RPT)], zrow_v.at[pl.ds(0, RPT)])
    pltpu.sync_copy(zrow_v.at[pl.ds(0, RPT)],
                    out_hbm.at[pl.ds(obase + NPAD + r0, RPT)])


def _degrees(src3, dst3):
    fn = pl.kernel(
        _deg_body,
        out_type=jax.ShapeDtypeStruct((NC * 2 * NPAD,), F32),
        mesh=_MESH,
        scratch_types=[
            pltpu.VMEM((NCHUNK, K), jnp.int32),
            pltpu.VMEM((NCHUNK, K), jnp.int32),
            pltpu.VMEM((K,), F32),
            pltpu.VMEM((RPT + 8,), F32),
            pltpu.VMEM_SHARED((NPAD,), F32),
            pltpu.VMEM_SHARED((NPAD,), F32),
            pltpu.SemaphoreType.DMA,
            pltpu.SemaphoreType.DMA,
        ],
    )
    return fn(src3, dst3)


def _mp_body(h_hbm, src_hbm, dst_hbm, out_hbm, sidx_all, didx0, didx1,
             rows0, rows1, acc, sem0, sem1, sem2, sem3):
    c = lax.axis_index("c")
    s = lax.axis_index("s")
    wid = s * NC + c

    @pl.loop(0, K)
    def _(r):
        @pl.loop(0, D, step=16)
        def _(c0):
            rows0[r, pl.ds(c0, 16)] = jnp.zeros((16,), F32)

    ebase = wid * EPTP
    pltpu.sync_copy(src_hbm.at[pl.ds(ebase, EPTP)], sidx_all)

    r0 = s * RPT

    @pl.loop(0, 4)
    def _(t):
        pltpu.sync_copy(rows0, acc.at[pl.ds(r0 + t * K, K)])

    pltpu.sync_copy(rows0.at[pl.ds(0, RPT - 4 * K)],
                    acc.at[pl.ds(r0 + 4 * K, RPT - 4 * K)])
    plsc.subcore_barrier()

    def dload(j, dbuf, sem):
        pltpu.async_copy(dst_hbm.at[pl.ds(ebase + j * K, K)], dbuf, sem)

    def dwait(j, dbuf, sem):
        pltpu.make_async_copy(dst_hbm.at[pl.ds(ebase + j * K, K)], dbuf, sem).wait()

    def gather(j, rbuf, sem):
        pltpu.async_copy(h_hbm.at[sidx_all.at[pl.ds(j * K, K)]], rbuf, sem)

    def gwait(j, rbuf, sem):
        pltpu.make_async_copy(h_hbm.at[sidx_all.at[pl.ds(j * K, K)]], rbuf, sem).wait()

    dload(0, didx0, sem2)
    gather(0, rows0, sem0)
    dload(1, didx1, sem3)
    gather(1, rows1, sem1)

    @pl.loop(0, NCHUNK // 2 - 1)
    def _(t):
        j = 2 * t
        dwait(j, didx0, sem2)
        gwait(j, rows0, sem0)
        pltpu.sync_copy(rows0, acc.at[didx0], add=True)
        dload(j + 2, didx0, sem2)
        gather(j + 2, rows0, sem0)
        dwait(j + 1, didx1, sem3)
        gwait(j + 1, rows1, sem1)
        pltpu.sync_copy(rows1, acc.at[didx1], add=True)
        dload(j + 3, didx1, sem3)
        gather(j + 3, rows1, sem1)

    jf = NCHUNK - 2
    dwait(jf, didx0, sem2)
    gwait(jf, rows0, sem0)
    pltpu.sync_copy(rows0, acc.at[didx0], add=True)
    dwait(jf + 1, didx1, sem3)
    gwait(jf + 1, rows1, sem1)
    pltpu.sync_copy(rows1, acc.at[didx1], add=True)

    plsc.subcore_barrier()
    pltpu.sync_copy(acc.at[pl.ds(r0, RPT)], out_hbm.at[c, pl.ds(r0, RPT)])


def _message_pass(h, src1, dst1):
    fn = pl.kernel(
        _mp_body,
        out_type=jax.ShapeDtypeStruct((NC, NPAD, D), F32),
        mesh=_MESH,
        scratch_types=[
            pltpu.VMEM((EPTP,), jnp.int32),
            pltpu.VMEM((K,), jnp.int32),
            pltpu.VMEM((K,), jnp.int32),
            pltpu.VMEM((K, D), F32),
            pltpu.VMEM((K, D), F32),
            pltpu.VMEM_SHARED((NPAD, D), F32),
            pltpu.SemaphoreType.DMA,
            pltpu.SemaphoreType.DMA,
            pltpu.SemaphoreType.DMA,
            pltpu.SemaphoreType.DMA,
        ],
    )
    return fn(h, src1, dst1)


def _ns_of(d_ref):
    return 1.0 / jnp.sqrt(jnp.maximum(d_ref[0, 0] + d_ref[1, 0], 1.0))


def _mm_plain(x, w):
    def body(x_ref, w_ref, o_ref):
        o_ref[...] = lax.dot_general(
            x_ref[...], w_ref[...],
            (((1,), (0,)), ((), ())), preferred_element_type=F32)

    return pl.pallas_call(
        body,
        grid=(N // BM,),
        in_specs=[
            pl.BlockSpec((BM, x.shape[1]), lambda i: (i, 0)),
            pl.BlockSpec(w.shape, lambda i: (0, 0)),
        ],
        out_specs=pl.BlockSpec((BM, w.shape[1]), lambda i: (i, 0)),
        out_shape=jax.ShapeDtypeStruct((NPAD, w.shape[1]), F32),
    )(x, w)


def _scale(u, degp):
    def body(u_ref, d_ref, o_ref):
        o_ref[...] = u_ref[...] * _ns_of(d_ref)

    return pl.pallas_call(
        body,
        grid=(N // BM,),
        in_specs=[
            pl.BlockSpec((BM, u.shape[1]), lambda i: (i, 0)),
            pl.BlockSpec((NC, 2, BM, 1), lambda i: (0, 0, i, 0)),
        ],
        out_specs=pl.BlockSpec((BM, u.shape[1]), lambda i: (i, 0)),
        out_shape=jax.ShapeDtypeStruct((NPAD, u.shape[1]), F32),
    )(u, degp)


def _mm_mid(p, degp, b, w):
    din = p.shape[2]

    def body(p_ref, d_ref, b_ref, w_ref, o_ref):
        nd = 1.0 / jnp.sqrt(jnp.maximum(d_ref[0, 1] + d_ref[1, 1], 1.0))
        h = (p_ref[0] + p_ref[1]) * nd + b_ref[...]
        h = jnp.maximum(h, 0.0)
        o_ref[...] = lax.dot_general(
            h * _ns_of(d_ref), w_ref[...],
            (((1,), (0,)), ((), ())), preferred_element_type=F32)

    return pl.pallas_call(
        body,
        grid=(N // BM,),
        in_specs=[
            pl.BlockSpec((NC, BM, din), lambda i: (0, i, 0)),
            pl.BlockSpec((NC, 2, BM, 1), lambda i: (0, 0, i, 0)),
            pl.BlockSpec((1, din), lambda i: (0, 0)),
            pl.BlockSpec(w.shape, lambda i: (0, 0)),
        ],
        out_specs=pl.BlockSpec((BM, w.shape[1]), lambda i: (i, 0)),
        out_shape=jax.ShapeDtypeStruct((NPAD, w.shape[1]), F32),
    )(p, degp, b, w)


def _final(p, degp, b):
    dout = p.shape[2]

    def body(p_ref, d_ref, b_ref, o_ref):
        nd = 1.0 / jnp.sqrt(jnp.maximum(d_ref[0, 1] + d_ref[1, 1], 1.0))
        o_ref[...] = (p_ref[0] + p_ref[1]) * nd + b_ref[...]

    return pl.pallas_call(
        body,
        grid=(N // BM,),
        in_specs=[
            pl.BlockSpec((NC, BM, dout), lambda i: (0, i, 0)),
            pl.BlockSpec((NC, 2, BM, 1), lambda i: (0, 0, i, 0)),
            pl.BlockSpec((1, dout), lambda i: (0, 0)),
        ],
        out_specs=pl.BlockSpec((BM, dout), lambda i: (i, 0)),
        out_shape=jax.ShapeDtypeStruct((N, dout), F32),
    )(p, degp, b)


def kernel(features, edge_index, W0, b0, W1, b1, W2, b2):
    ei = edge_index.reshape(2, NW, EPT)
    pad = jnp.full((2, NW, EPTP - EPT), NPAD - 1, jnp.int32)
    eip = jnp.concatenate([ei, pad], axis=2)
    src3 = eip[0].reshape(NW, NCHUNK, K)
    dst3 = eip[1].reshape(NW, NCHUNK, K)
    src1 = eip[0].reshape(NW * EPTP)
    dst1 = eip[1].reshape(NW * EPTP)

    w2p = jnp.pad(W2, ((0, 0), (0, 88)))
    b2p = jnp.pad(b2, (0, 88))

    u0 = _mm_plain(features, W0)
    degp = _degrees(src3, dst3)
    degp4 = degp.reshape(NC, 2, NPAD, 1)

    h0 = _scale(u0, degp4)
    p0 = _message_pass(h0, src1, dst1)
    h1 = _mm_mid(p0, degp4, b0.reshape(1, -1), W1)
    p1 = _message_pass(h1, src1, dst1)
    h2 = _mm_mid(p1, degp4, b1.reshape(1, -1), w2p)
    p2 = _message_pass(h2, src1, dst1)
    out = _final(p2, degp4, b2p.reshape(1, -1))
    return out[:, :40]


# speedup vs baseline: 2.3195x; 2.3195x over previous
import jax
import jax.numpy as jnp
from jax import lax
from jax.experimental import pallas as pl
from jax.experimental.pallas import tpu as pltpu
from jax.experimental.pallas import tpu_sc as plsc

N = 10000
E = 320000
NC = 2
NS = 16
NW = NC * NS
K = 128
KM = 80
EPT = E // NW
NCHUNK = 80
NCHUNKM = 128
EPTP = NCHUNK * K
NPAD = 10112
RPT = NPAD // NS
D = 128
BM = 400
F32 = jnp.float32

_MESH = plsc.VectorSubcoreMesh(core_axis_name="c", subcore_axis_name="s")


def _deg_body(src_hbm, dst_hbm, out_hbm, sidx_all, didx_all, ones_v, zrow_v,
              acc_out, acc_in, sem0, sem1):
    c = lax.axis_index("c")
    s = lax.axis_index("s")
    wid = s * NC + c

    @pl.loop(0, K, step=16)
    def _(i):
        ones_v[pl.ds(i, 16)] = jnp.ones((16,), F32)

    @pl.loop(0, RPT + 8, step=16)
    def _(i):
        zrow_v[pl.ds(i, 16)] = jnp.zeros((16,), F32)

    pltpu.sync_copy(src_hbm.at[wid], sidx_all)
    pltpu.sync_copy(dst_hbm.at[wid], didx_all)

    r0 = s * RPT
    pltpu.sync_copy(zrow_v.at[pl.ds(0, RPT)], acc_out.at[pl.ds(r0, RPT)])
    pltpu.sync_copy(zrow_v.at[pl.ds(0, RPT)], acc_in.at[pl.ds(r0, RPT)])
    plsc.subcore_barrier()

    def fire(j):
        pltpu.async_copy(ones_v, acc_out.at[sidx_all.at[j]], sem0, add=True)
        pltpu.async_copy(ones_v, acc_in.at[didx_all.at[j]], sem1, add=True)

    def drain(j):
        pltpu.make_async_copy(ones_v, acc_out.at[sidx_all.at[j]], sem0).wait()
        pltpu.make_async_copy(ones_v, acc_in.at[didx_all.at[j]], sem1).wait()

    fire(0)

    @pl.loop(1, NCHUNK)
    def _(j):
        fire(j)
        drain(j - 1)

    drain(NCHUNK - 1)

    plsc.subcore_barrier()
    obase = c * 2 * NPAD

    pltpu.sync_copy(acc_out.at[pl.ds(r0, RPT)], zrow_v.at[pl.ds(0, RPT)])
    pltpu.sync_copy(zrow_v.at[pl.ds(0, RPT)], out_hbm.at[pl.ds(obase + r0, RPT)])
    pltpu.sync_copy(acc_in.at[pl.ds(r0, RPT)], zrow_v.at[pl.ds(0, RPT)])
    pltpu.sync_copy(zrow_v.at[pl.ds(0, RPT)],
                    out_hbm.at[pl.ds(obase + NPAD + r0, RPT)])


def _degrees(src3, dst3):
    fn = pl.kernel(
        _deg_body,
        out_type=jax.ShapeDtypeStruct((NC * 2 * NPAD,), F32),
        mesh=_MESH,
        scratch_types=[
            pltpu.VMEM((NCHUNK, K), jnp.int32),
            pltpu.VMEM((NCHUNK, K), jnp.int32),
            pltpu.VMEM((K,), F32),
            pltpu.VMEM((RPT + 8,), F32),
            pltpu.VMEM_SHARED((NPAD,), F32),
            pltpu.VMEM_SHARED((NPAD,), F32),
            pltpu.SemaphoreType.DMA,
            pltpu.SemaphoreType.DMA,
        ],
    )
    return fn(src3, dst3)


def _mp_body(h_hbm, src_hbm, dst_hbm, out_hbm, sidx_all, didx0, didx1,
             rows0, rows1, acc, sem0, sem1, sem2, sem3):
    c = lax.axis_index("c")
    s = lax.axis_index("s")
    wid = s * NC + c

    @pl.loop(0, KM)
    def _(r):
        @pl.loop(0, D, step=16)
        def _(c0):
            rows0[r, pl.ds(c0, 16)] = jnp.zeros((16,), F32)

    ebase = wid * EPTP
    pltpu.sync_copy(src_hbm.at[pl.ds(ebase, EPTP)], sidx_all)

    r0 = s * RPT

    @pl.loop(0, RPT // KM)
    def _(t):
        pltpu.sync_copy(rows0, acc.at[pl.ds(r0 + t * KM, KM)])

    pltpu.sync_copy(rows0.at[pl.ds(0, RPT % KM)],
                    acc.at[pl.ds(r0 + (RPT // KM) * KM, RPT % KM)])
    plsc.subcore_barrier()

    def dload(j, dbuf, sem):
        pltpu.async_copy(dst_hbm.at[pl.ds(ebase + j * KM, KM)], dbuf, sem)

    def dwait(j, dbuf, sem):
        pltpu.make_async_copy(dst_hbm.at[pl.ds(ebase + j * KM, KM)], dbuf, sem).wait()

    def gather(j, rbuf, sem):
        pltpu.async_copy(h_hbm.at[sidx_all.at[pl.ds(j * KM, KM)]], rbuf, sem)

    def gwait(j, rbuf, sem):
        pltpu.make_async_copy(h_hbm.at[sidx_all.at[pl.ds(j * KM, KM)]], rbuf, sem).wait()

    dload(0, didx0, sem2)
    gather(0, rows0, sem0)
    dload(1, didx1, sem3)
    gather(1, rows1, sem1)

    @pl.loop(0, NCHUNKM // 2 - 1)
    def _(t):
        j = 2 * t
        dwait(j, didx0, sem2)
        gwait(j, rows0, sem0)
        pltpu.sync_copy(rows0, acc.at[didx0], add=True)
        dload(j + 2, didx0, sem2)
        gather(j + 2, rows0, sem0)
        dwait(j + 1, didx1, sem3)
        gwait(j + 1, rows1, sem1)
        pltpu.sync_copy(rows1, acc.at[didx1], add=True)
        dload(j + 3, didx1, sem3)
        gather(j + 3, rows1, sem1)

    jf = NCHUNKM - 2
    dwait(jf, didx0, sem2)
    gwait(jf, rows0, sem0)
    pltpu.sync_copy(rows0, acc.at[didx0], add=True)
    dwait(jf + 1, didx1, sem3)
    gwait(jf + 1, rows1, sem1)
    pltpu.sync_copy(rows1, acc.at[didx1], add=True)

    plsc.subcore_barrier()
    pltpu.sync_copy(acc.at[pl.ds(r0, RPT)], out_hbm.at[c, pl.ds(r0, RPT)])


def _message_pass(h, src1, dst1):
    fn = pl.kernel(
        _mp_body,
        out_type=jax.ShapeDtypeStruct((NC, NPAD, D), F32),
        mesh=_MESH,
        scratch_types=[
            pltpu.VMEM((EPTP,), jnp.int32),
            pltpu.VMEM((KM,), jnp.int32),
            pltpu.VMEM((KM,), jnp.int32),
            pltpu.VMEM((KM, D), F32),
            pltpu.VMEM((KM, D), F32),
            pltpu.VMEM_SHARED((NPAD, D), F32),
            pltpu.SemaphoreType.DMA,
            pltpu.SemaphoreType.DMA,
            pltpu.SemaphoreType.DMA,
            pltpu.SemaphoreType.DMA,
        ],
    )
    return fn(h, src1, dst1)


def _ns_of(d_ref):
    return 1.0 / jnp.sqrt(jnp.maximum(d_ref[0, 0] + d_ref[1, 0], 1.0))


def _mm_plain(x, w):
    def body(x_ref, w_ref, o_ref):
        o_ref[...] = lax.dot_general(
            x_ref[...], w_ref[...],
            (((1,), (0,)), ((), ())), preferred_element_type=F32)

    return pl.pallas_call(
        body,
        grid=(N // BM,),
        in_specs=[
            pl.BlockSpec((BM, x.shape[1]), lambda i: (i, 0)),
            pl.BlockSpec(w.shape, lambda i: (0, 0)),
        ],
        out_specs=pl.BlockSpec((BM, w.shape[1]), lambda i: (i, 0)),
        out_shape=jax.ShapeDtypeStruct((NPAD, w.shape[1]), F32),
    )(x, w)


def _scale(u, degp):
    def body(u_ref, d_ref, o_ref):
        o_ref[...] = u_ref[...] * _ns_of(d_ref)

    return pl.pallas_call(
        body,
        grid=(N // BM,),
        in_specs=[
            pl.BlockSpec((BM, u.shape[1]), lambda i: (i, 0)),
            pl.BlockSpec((NC, 2, BM, 1), lambda i: (0, 0, i, 0)),
        ],
        out_specs=pl.BlockSpec((BM, u.shape[1]), lambda i: (i, 0)),
        out_shape=jax.ShapeDtypeStruct((NPAD, u.shape[1]), F32),
    )(u, degp)


def _mm_mid(p, degp, b, w):
    din = p.shape[2]

    def body(p_ref, d_ref, b_ref, w_ref, o_ref):
        nd = 1.0 / jnp.sqrt(jnp.maximum(d_ref[0, 1] + d_ref[1, 1], 1.0))
        h = (p_ref[0] + p_ref[1]) * nd + b_ref[...]
        h = jnp.maximum(h, 0.0)
        o_ref[...] = lax.dot_general(
            h * _ns_of(d_ref), w_ref[...],
            (((1,), (0,)), ((), ())), preferred_element_type=F32)

    return pl.pallas_call(
        body,
        grid=(N // BM,),
        in_specs=[
            pl.BlockSpec((NC, BM, din), lambda i: (0, i, 0)),
            pl.BlockSpec((NC, 2, BM, 1), lambda i: (0, 0, i, 0)),
            pl.BlockSpec((1, din), lambda i: (0, 0)),
            pl.BlockSpec(w.shape, lambda i: (0, 0)),
        ],
        out_specs=pl.BlockSpec((BM, w.shape[1]), lambda i: (i, 0)),
        out_shape=jax.ShapeDtypeStruct((NPAD, w.shape[1]), F32),
    )(p, degp, b, w)


def _final(p, degp, b):
    dout = p.shape[2]

    def body(p_ref, d_ref, b_ref, o_ref):
        nd = 1.0 / jnp.sqrt(jnp.maximum(d_ref[0, 1] + d_ref[1, 1], 1.0))
        o_ref[...] = (p_ref[0] + p_ref[1]) * nd + b_ref[...]

    return pl.pallas_call(
        body,
        grid=(N // BM,),
        in_specs=[
            pl.BlockSpec((NC, BM, dout), lambda i: (0, i, 0)),
            pl.BlockSpec((NC, 2, BM, 1), lambda i: (0, 0, i, 0)),
            pl.BlockSpec((1, dout), lambda i: (0, 0)),
        ],
        out_specs=pl.BlockSpec((BM, dout), lambda i: (i, 0)),
        out_shape=jax.ShapeDtypeStruct((N, dout), F32),
    )(p, degp, b)


def kernel(features, edge_index, W0, b0, W1, b1, W2, b2):
    ei = edge_index.reshape(2, NW, EPT)
    pad = jnp.broadcast_to((N + jnp.arange(NW, dtype=jnp.int32))[None, :, None],
                           (2, NW, EPTP - EPT))
    eip = jnp.concatenate([ei, pad], axis=2)
    src3 = eip[0].reshape(NW, NCHUNK, K)
    dst3 = eip[1].reshape(NW, NCHUNK, K)
    src1 = eip[0].reshape(NW * EPTP)
    dst1 = eip[1].reshape(NW * EPTP)

    w2p = jnp.pad(W2, ((0, 0), (0, 88)))
    b2p = jnp.pad(b2, (0, 88))

    u0 = _mm_plain(features, W0)
    degp = _degrees(src3, dst3)
    degp4 = degp.reshape(NC, 2, NPAD, 1)

    h0 = _scale(u0, degp4)
    p0 = _message_pass(h0, src1, dst1)
    h1 = _mm_mid(p0, degp4, b0.reshape(1, -1), W1)
    p1 = _message_pass(h1, src1, dst1)
    h2 = _mm_mid(p1, degp4, b1.reshape(1, -1), w2p)
    p2 = _message_pass(h2, src1, dst1)
    out = _final(p2, degp4, b2p.reshape(1, -1))
    return out[:, :40]


# speedup vs baseline: 2.6500x; 1.1425x over previous
import jax
import jax.numpy as jnp
from jax import lax
from jax.experimental import pallas as pl
from jax.experimental.pallas import tpu as pltpu
from jax.experimental.pallas import tpu_sc as plsc

N = 10000
E = 320000
NC = 2
NS = 16
NW = NC * NS
K = 128
KM = 80
EPT = E // NW
NCHUNK = 80
NCHUNKM = EPT // KM
EPTP = NCHUNK * K
NPAD = 10112
RPT = NPAD // NS
D = 128
BM = 400
F32 = jnp.float32

_MESH = plsc.VectorSubcoreMesh(core_axis_name="c", subcore_axis_name="s")


def _deg_body(src_hbm, dst_hbm, out_hbm, sidx_all, didx_all, ones_v, zrow_v,
              acc_out, acc_in, sem0, sem1):
    c = lax.axis_index("c")
    s = lax.axis_index("s")
    wid = s * NC + c

    @pl.loop(0, K, step=16)
    def _(i):
        ones_v[pl.ds(i, 16)] = jnp.ones((16,), F32)

    @pl.loop(0, RPT + 8, step=16)
    def _(i):
        zrow_v[pl.ds(i, 16)] = jnp.zeros((16,), F32)

    pltpu.sync_copy(src_hbm.at[wid], sidx_all)
    pltpu.sync_copy(dst_hbm.at[wid], didx_all)

    r0 = s * RPT
    pltpu.sync_copy(zrow_v.at[pl.ds(0, RPT)], acc_out.at[pl.ds(r0, RPT)])
    pltpu.sync_copy(zrow_v.at[pl.ds(0, RPT)], acc_in.at[pl.ds(r0, RPT)])
    plsc.subcore_barrier()

    def fire(j):
        pltpu.async_copy(ones_v, acc_out.at[sidx_all.at[j]], sem0, add=True)
        pltpu.async_copy(ones_v, acc_in.at[didx_all.at[j]], sem1, add=True)

    def drain(j):
        pltpu.make_async_copy(ones_v, acc_out.at[sidx_all.at[j]], sem0).wait()
        pltpu.make_async_copy(ones_v, acc_in.at[didx_all.at[j]], sem1).wait()

    fire(0)

    @pl.loop(1, NCHUNK)
    def _(j):
        fire(j)
        drain(j - 1)

    drain(NCHUNK - 1)

    plsc.subcore_barrier()
    obase = c * 2 * NPAD

    pltpu.sync_copy(acc_out.at[pl.ds(r0, RPT)], zrow_v.at[pl.ds(0, RPT)])
    pltpu.sync_copy(zrow_v.at[pl.ds(0, RPT)], out_hbm.at[pl.ds(obase + r0, RPT)])
    pltpu.sync_copy(acc_in.at[pl.ds(r0, RPT)], zrow_v.at[pl.ds(0, RPT)])
    pltpu.sync_copy(zrow_v.at[pl.ds(0, RPT)],
                    out_hbm.at[pl.ds(obase + NPAD + r0, RPT)])


def _degrees(src3, dst3):
    fn = pl.kernel(
        _deg_body,
        out_type=jax.ShapeDtypeStruct((NC * 2 * NPAD,), F32),
        mesh=_MESH,
        scratch_types=[
            pltpu.VMEM((NCHUNK, K), jnp.int32),
            pltpu.VMEM((NCHUNK, K), jnp.int32),
            pltpu.VMEM((K,), F32),
            pltpu.VMEM((RPT + 8,), F32),
            pltpu.VMEM_SHARED((NPAD,), F32),
            pltpu.VMEM_SHARED((NPAD,), F32),
            pltpu.SemaphoreType.DMA,
            pltpu.SemaphoreType.DMA,
        ],
    )
    return fn(src3, dst3)


def _mp_body(h_hbm, src_hbm, dst_hbm, out_hbm, sidx_all, didx0, didx1,
             rows0, rows1, acc, sem0, sem1, sem2, sem3):
    c = lax.axis_index("c")
    s = lax.axis_index("s")
    wid = s * NC + c

    @pl.loop(0, KM)
    def _(r):
        @pl.loop(0, D, step=16)
        def _(c0):
            rows0[r, pl.ds(c0, 16)] = jnp.zeros((16,), F32)

    ebase = wid * EPT
    pltpu.sync_copy(src_hbm.at[pl.ds(ebase, EPT)], sidx_all)

    r0 = s * RPT

    @pl.loop(0, RPT // KM)
    def _(t):
        pltpu.sync_copy(rows0, acc.at[pl.ds(r0 + t * KM, KM)])

    pltpu.sync_copy(rows0.at[pl.ds(0, RPT % KM)],
                    acc.at[pl.ds(r0 + (RPT // KM) * KM, RPT % KM)])
    plsc.subcore_barrier()

    def dload(j, dbuf, sem):
        pltpu.async_copy(dst_hbm.at[pl.ds(ebase + j * KM, KM)], dbuf, sem)

    def dwait(j, dbuf, sem):
        pltpu.make_async_copy(dst_hbm.at[pl.ds(ebase + j * KM, KM)], dbuf, sem).wait()

    def gather(j, rbuf, sem):
        pltpu.async_copy(h_hbm.at[sidx_all.at[pl.ds(j * KM, KM)]], rbuf, sem)

    def gwait(j, rbuf, sem):
        pltpu.make_async_copy(h_hbm.at[sidx_all.at[pl.ds(j * KM, KM)]], rbuf, sem).wait()

    dload(0, didx0, sem2)
    gather(0, rows0, sem0)
    dload(1, didx1, sem3)
    gather(1, rows1, sem1)

    @pl.loop(0, (NCHUNKM - 3) // 2)
    def _(t):
        j = 2 * t
        dwait(j, didx0, sem2)
        gwait(j, rows0, sem0)
        pltpu.sync_copy(rows0, acc.at[didx0], add=True)
        dload(j + 2, didx0, sem2)
        gather(j + 2, rows0, sem0)
        dwait(j + 1, didx1, sem3)
        gwait(j + 1, rows1, sem1)
        pltpu.sync_copy(rows1, acc.at[didx1], add=True)
        dload(j + 3, didx1, sem3)
        gather(j + 3, rows1, sem1)

    jf = NCHUNKM - 3
    dwait(jf, didx0, sem2)
    gwait(jf, rows0, sem0)
    pltpu.sync_copy(rows0, acc.at[didx0], add=True)
    dload(jf + 2, didx0, sem2)
    gather(jf + 2, rows0, sem0)
    dwait(jf + 1, didx1, sem3)
    gwait(jf + 1, rows1, sem1)
    pltpu.sync_copy(rows1, acc.at[didx1], add=True)
    dwait(jf + 2, didx0, sem2)
    gwait(jf + 2, rows0, sem0)
    pltpu.sync_copy(rows0, acc.at[didx0], add=True)

    plsc.subcore_barrier()
    pltpu.sync_copy(acc.at[pl.ds(r0, RPT)], out_hbm.at[c, pl.ds(r0, RPT)])


def _message_pass(h, src1, dst1):
    fn = pl.kernel(
        _mp_body,
        out_type=jax.ShapeDtypeStruct((NC, NPAD, D), F32),
        mesh=_MESH,
        scratch_types=[
            pltpu.VMEM((EPT,), jnp.int32),
            pltpu.VMEM((KM,), jnp.int32),
            pltpu.VMEM((KM,), jnp.int32),
            pltpu.VMEM((KM, D), F32),
            pltpu.VMEM((KM, D), F32),
            pltpu.VMEM_SHARED((NPAD, D), F32),
            pltpu.SemaphoreType.DMA,
            pltpu.SemaphoreType.DMA,
            pltpu.SemaphoreType.DMA,
            pltpu.SemaphoreType.DMA,
        ],
    )
    return fn(h, src1, dst1)


def _ns_of(d_ref):
    return 1.0 / jnp.sqrt(jnp.maximum(d_ref[0, 0] + d_ref[1, 0], 1.0))


def _mm_first(x, degp, w):
    def body(x_ref, d_ref, w_ref, o_ref):
        o_ref[...] = lax.dot_general(
            x_ref[...] * _ns_of(d_ref), w_ref[...],
            (((1,), (0,)), ((), ())), preferred_element_type=F32)

    return pl.pallas_call(
        body,
        grid=(N // BM,),
        in_specs=[
            pl.BlockSpec((BM, x.shape[1]), lambda i: (i, 0)),
            pl.BlockSpec((NC, 2, BM, 1), lambda i: (0, 0, i, 0)),
            pl.BlockSpec(w.shape, lambda i: (0, 0)),
        ],
        out_specs=pl.BlockSpec((BM, w.shape[1]), lambda i: (i, 0)),
        out_shape=jax.ShapeDtypeStruct((NPAD, w.shape[1]), F32),
    )(x, degp, w)


def _mm_mid(p, degp, b, w):
    din = p.shape[2]

    def body(p_ref, d_ref, b_ref, w_ref, o_ref):
        nd = 1.0 / jnp.sqrt(jnp.maximum(d_ref[0, 1] + d_ref[1, 1], 1.0))
        h = (p_ref[0] + p_ref[1]) * nd + b_ref[...]
        h = jnp.maximum(h, 0.0)
        o_ref[...] = lax.dot_general(
            h * _ns_of(d_ref), w_ref[...],
            (((1,), (0,)), ((), ())), preferred_element_type=F32)

    return pl.pallas_call(
        body,
        grid=(N // BM,),
        in_specs=[
            pl.BlockSpec((NC, BM, din), lambda i: (0, i, 0)),
            pl.BlockSpec((NC, 2, BM, 1), lambda i: (0, 0, i, 0)),
            pl.BlockSpec((1, din), lambda i: (0, 0)),
            pl.BlockSpec(w.shape, lambda i: (0, 0)),
        ],
        out_specs=pl.BlockSpec((BM, w.shape[1]), lambda i: (i, 0)),
        out_shape=jax.ShapeDtypeStruct((NPAD, w.shape[1]), F32),
    )(p, degp, b, w)


def _final(p, degp, b):
    dout = p.shape[2]

    def body(p_ref, d_ref, b_ref, o_ref):
        nd = 1.0 / jnp.sqrt(jnp.maximum(d_ref[0, 1] + d_ref[1, 1], 1.0))
        o_ref[...] = (p_ref[0] + p_ref[1]) * nd + b_ref[...]

    return pl.pallas_call(
        body,
        grid=(N // BM,),
        in_specs=[
            pl.BlockSpec((NC, BM, dout), lambda i: (0, i, 0)),
            pl.BlockSpec((NC, 2, BM, 1), lambda i: (0, 0, i, 0)),
            pl.BlockSpec((1, dout), lambda i: (0, 0)),
        ],
        out_specs=pl.BlockSpec((BM, dout), lambda i: (i, 0)),
        out_shape=jax.ShapeDtypeStruct((N, dout), F32),
    )(p, degp, b)


def kernel(features, edge_index, W0, b0, W1, b1, W2, b2):
    ei = edge_index.reshape(2, NW, EPT)
    pad = jnp.broadcast_to((N + jnp.arange(NW, dtype=jnp.int32))[None, :, None],
                           (2, NW, EPTP - EPT))
    eip = jnp.concatenate([ei, pad], axis=2)
    src3 = eip[0].reshape(NW, NCHUNK, K)
    dst3 = eip[1].reshape(NW, NCHUNK, K)
    src1 = edge_index[0]
    dst1 = edge_index[1]

    w2p = jnp.pad(W2, ((0, 0), (0, 88)))
    b2p = jnp.pad(b2, (0, 88))

    degp = _degrees(src3, dst3)
    degp4 = degp.reshape(NC, 2, NPAD, 1)

    h0 = _mm_first(features, degp4, W0)
    p0 = _message_pass(h0, src1, dst1)
    h1 = _mm_mid(p0, degp4, b0.reshape(1, -1), W1)
    p1 = _message_pass(h1, src1, dst1)
    h2 = _mm_mid(p1, degp4, b1.reshape(1, -1), w2p)
    p2 = _message_pass(h2, src1, dst1)
    out = _final(p2, degp4, b2p.reshape(1, -1))
    return out[:, :40]
